# Initial kernel scaffold; baseline (speedup 1.0000x reference)
import jax
import jax.numpy as jnp
from jax.experimental import pallas as pl


def kernel(input_, W, A_buffer, B_buffer, weight_indices):
    raise NotImplementedError("write your pallas kernel here")


# speedup vs baseline: 4.1134x; 4.1134x over previous
import functools

import jax
import jax.numpy as jnp
from jax.experimental import pallas as pl
from jax.experimental.pallas import tpu as pltpu

T, D, E, RANK, B_DIM = 4096, 2048, 8, 16, 4096
ER2 = E * 2 * RANK

TM = 1024
TN = 512


def _fused_kernel(idx_ref, x_ref, w_ref, aall_ref, bbig_ref, out_ref, xa_ref):
    j = pl.program_id(1)

    @pl.when(j == 0)
    def _():
        xa = jnp.dot(x_ref[...], aall_ref[...],
                     preferred_element_type=jnp.float32)
        col_expert = jax.lax.broadcasted_iota(jnp.int32, (TM, ER2), 1) // (2 * RANK)
        xa_ref[...] = jnp.where(col_expert == idx_ref[...], xa, 0.0)

    out_ref[...] = (
        jnp.dot(x_ref[...], w_ref[...], preferred_element_type=jnp.float32)
        + jnp.dot(xa_ref[...], bbig_ref[...], preferred_element_type=jnp.float32)
    )


@functools.partial(jax.jit, static_argnames=())
def kernel(input_, W, A_buffer, B_buffer, weight_indices):
    A_all = A_buffer.transpose(1, 0, 2).reshape(D, ER2)
    B4 = B_buffer.reshape(E, RANK, 2, B_DIM)
    B_big = jnp.einsum('ergn,hg->ehrgn', B4, jnp.eye(2, dtype=B4.dtype))
    B_big = B_big.reshape(ER2, 2 * B_DIM)

    idx2d = weight_indices.astype(jnp.int32).reshape(T, 1)

    ni, nj = T // TM, (2 * B_DIM) // TN
    out = pl.pallas_call(
        _fused_kernel,
        grid=(ni, nj),
        in_specs=[
            pl.BlockSpec((TM, 1), lambda i, j: (i, 0)),
            pl.BlockSpec((TM, D), lambda i, j: (i, 0)),
            pl.BlockSpec((D, TN), lambda i, j: (0, j)),
            pl.BlockSpec((D, ER2), lambda i, j: (0, 0)),
            pl.BlockSpec((ER2, TN), lambda i, j: (0, j)),
        ],
        out_specs=pl.BlockSpec((TM, TN), lambda i, j: (i, j)),
        out_shape=jax.ShapeDtypeStruct((T, 2 * B_DIM), jnp.float32),
        scratch_shapes=[pltpu.VMEM((TM, ER2), jnp.float32)],
    )(idx2d, input_, W, A_all, B_big)
    return out


# speedup vs baseline: 4.7749x; 1.1608x over previous
import functools

import jax
import jax.numpy as jnp
from jax.experimental import pallas as pl
from jax.experimental.pallas import tpu as pltpu

T, D, E, RANK, B_DIM = 4096, 2048, 8, 16, 4096
ER = E * RANK
N_OUT = 2 * B_DIM

TM = 1024
TN = 512
NJH = B_DIM // TN


def _fused_kernel(idx_ref, x_ref, w_ref, ahall_ref, bres_ref, out_ref, xa_ref):
    j = pl.program_id(1)

    @pl.when(j == 0)
    def _():
        xa = jnp.dot(x_ref[...], ahall_ref[...],
                     preferred_element_type=jnp.float32)
        col = jax.lax.broadcasted_iota(jnp.int32, (TM, 2 * ER), 1)
        col_expert = (col // RANK) % E
        xa = jnp.where(col_expert == idx_ref[...], xa, 0.0)
        xa_ref[0] = xa[:, :ER]
        xa_ref[1] = xa[:, ER:]

    h = j // NJH
    out_ref[...] = (
        jnp.dot(x_ref[...], w_ref[...], preferred_element_type=jnp.float32)
        + jnp.dot(xa_ref[h], bres_ref[...], preferred_element_type=jnp.float32)
    )


@functools.partial(jax.jit, static_argnames=())
def kernel(input_, W, A_buffer, B_buffer, weight_indices):
    A_hall = (A_buffer.reshape(E, D, 2, RANK)
              .transpose(1, 2, 0, 3).reshape(D, 2 * ER))
    B_res = B_buffer.reshape(ER, N_OUT)

    idx2d = weight_indices.astype(jnp.int32).reshape(T, 1)

    ni, nj = T // TM, N_OUT // TN
    out = pl.pallas_call(
        _fused_kernel,
        grid=(ni, nj),
        in_specs=[
            pl.BlockSpec((TM, 1), lambda i, j: (i, 0)),
            pl.BlockSpec((TM, D), lambda i, j: (i, 0)),
            pl.BlockSpec((D, TN), lambda i, j: (0, j)),
            pl.BlockSpec((D, 2 * ER), lambda i, j: (0, 0)),
            pl.BlockSpec((ER, TN), lambda i, j: (0, j)),
        ],
        out_specs=pl.BlockSpec((TM, TN), lambda i, j: (i, j)),
        out_shape=jax.ShapeDtypeStruct((T, N_OUT), jnp.float32),
        scratch_shapes=[pltpu.VMEM((2, TM, ER), jnp.float32)],
    )(idx2d, input_, W, A_hall, B_res)
    return out
